# baseline (speedup 1.0000x reference)
import functools

import jax
import jax.numpy as jnp
from jax.experimental import pallas as pl
from jax.experimental.pallas import tpu as pltpu

_EPS = 1e-5


def _mm_stats_body(x_ref, w_ref, b_ref, h_ref, st_ref):
    h = jnp.dot(x_ref[...], w_ref[...], preferred_element_type=jnp.float32)
    h = h + b_ref[...]
    h_ref[...] = h
    s = jnp.sum(h, axis=0, keepdims=True)
    sq = jnp.sum(h * h, axis=0, keepdims=True)
    st_ref[...] = jnp.concatenate([s, sq], axis=0)[None]


def _bn_from_partials(st_ref, g_ref, be_ref, n_rows):
    st = jnp.sum(st_ref[...], axis=0)
    inv_n = 1.0 / n_rows
    mean = st[0:1, :] * inv_n
    var = st[1:2, :] * inv_n - mean * mean
    scale = jax.lax.rsqrt(var + _EPS) * g_ref[...]
    shift = be_ref[...] - mean * scale
    return scale, shift


def _bn_mm_stats_body(n_rows, h_ref, st_in_ref, g_ref, be_ref, w_ref, b_ref,
                      o_ref, st_out_ref):
    scale, shift = _bn_from_partials(st_in_ref, g_ref, be_ref, n_rows)
    a = jnp.maximum(h_ref[...] * scale + shift, 0.0)
    h2 = jnp.dot(a, w_ref[...], preferred_element_type=jnp.float32)
    h2 = h2 + b_ref[...]
    o_ref[...] = h2
    s = jnp.sum(h2, axis=0, keepdims=True)
    sq = jnp.sum(h2 * h2, axis=0, keepdims=True)
    st_out_ref[...] = jnp.concatenate([s, sq], axis=0)[None]


def _bn_mm_body(n_rows, h_ref, st_in_ref, g_ref, be_ref, w_ref, b_ref, o_ref):
    scale, shift = _bn_from_partials(st_in_ref, g_ref, be_ref, n_rows)
    a = jnp.maximum(h_ref[...] * scale + shift, 0.0)
    h2 = jnp.dot(a, w_ref[...], preferred_element_type=jnp.float32)
    o_ref[...] = h2 + b_ref[...]


def _row_spec(blk, d):
    return pl.BlockSpec((blk, d), lambda i: (i, 0))


def _full_spec(shape):
    nd = len(shape)
    return pl.BlockSpec(shape, lambda i: (0,) * nd)


def _part_spec(f):
    return pl.BlockSpec((1, 2, f), lambda i: (i, 0, 0))


def kernel(input, W1, b1, g1, be1, W2, b2, g2, be2, W3, b3):
    n, d = input.shape
    f = W1.shape[0]
    blk = 2000
    nblk = n // blk
    grid = (nblk,)
    params = pltpu.CompilerParams(dimension_semantics=("parallel",))

    w1t = W1.T
    w2t = W2.T
    w3t = W3.T
    b1r = b1.reshape(1, f)
    b2r = b2.reshape(1, f)
    b3r = b3.reshape(1, f)
    g1r = g1.reshape(1, f)
    g2r = g2.reshape(1, f)
    be1r = be1.reshape(1, f)
    be2r = be2.reshape(1, f)

    h1, st1 = pl.pallas_call(
        _mm_stats_body,
        grid=grid,
        in_specs=[_row_spec(blk, d), _full_spec((d, f)), _full_spec((1, f))],
        out_specs=[_row_spec(blk, f), _part_spec(f)],
        out_shape=[
            jax.ShapeDtypeStruct((n, f), jnp.float32),
            jax.ShapeDtypeStruct((nblk, 2, f), jnp.float32),
        ],
        compiler_params=params,
    )(input, w1t, b1r)

    h2, st2 = pl.pallas_call(
        functools.partial(_bn_mm_stats_body, float(n)),
        grid=grid,
        in_specs=[_row_spec(blk, f), _full_spec((nblk, 2, f)),
                  _full_spec((1, f)), _full_spec((1, f)),
                  _full_spec((f, f)), _full_spec((1, f))],
        out_specs=[_row_spec(blk, f), _part_spec(f)],
        out_shape=[
            jax.ShapeDtypeStruct((n, f), jnp.float32),
            jax.ShapeDtypeStruct((nblk, 2, f), jnp.float32),
        ],
        compiler_params=params,
    )(h1, st1, g1r, be1r, w2t, b2r)

    out = pl.pallas_call(
        functools.partial(_bn_mm_body, float(n)),
        grid=grid,
        in_specs=[_row_spec(blk, f), _full_spec((nblk, 2, f)),
                  _full_spec((1, f)), _full_spec((1, f)),
                  _full_spec((f, f)), _full_spec((1, f))],
        out_specs=_row_spec(blk, f),
        out_shape=jax.ShapeDtypeStruct((n, f), jnp.float32),
        compiler_params=params,
    )(h2, st2, g2r, be2r, w3t, b3r)

    return out


# speedup vs baseline: 1.7282x; 1.7282x over previous
import functools

import jax
import jax.numpy as jnp
from jax.experimental import pallas as pl
from jax.experimental.pallas import tpu as pltpu

_EPS = 1e-5
_DN = (((1,), (1,)), ((), ()))


def _mm_stats_body(x_ref, w_ref, b_ref, h_ref, st_ref):
    xb = x_ref[...].astype(jnp.bfloat16)
    wb = w_ref[...].astype(jnp.bfloat16)
    h = jax.lax.dot_general(xb, wb, _DN, preferred_element_type=jnp.float32)
    h = h + b_ref[...]
    h_ref[...] = h.astype(jnp.bfloat16)
    s = jnp.sum(h, axis=0, keepdims=True)
    sq = jnp.sum(h * h, axis=0, keepdims=True)
    st_ref[...] = jnp.concatenate([s, sq], axis=0)[None]


def _bn_from_partials(st_ref, g_ref, be_ref, n_rows):
    st = jnp.sum(st_ref[...], axis=0)
    inv_n = 1.0 / n_rows
    mean = st[0:1, :] * inv_n
    var = st[1:2, :] * inv_n - mean * mean
    scale = jax.lax.rsqrt(var + _EPS) * g_ref[...]
    shift = be_ref[...] - mean * scale
    return scale, shift


def _bn_mm_stats_body(n_rows, h_ref, st_in_ref, g_ref, be_ref, w_ref, b_ref,
                      o_ref, st_out_ref):
    scale, shift = _bn_from_partials(st_in_ref, g_ref, be_ref, n_rows)
    a = jnp.maximum(h_ref[...].astype(jnp.float32) * scale + shift, 0.0)
    ab = a.astype(jnp.bfloat16)
    wb = w_ref[...].astype(jnp.bfloat16)
    h2 = jax.lax.dot_general(ab, wb, _DN, preferred_element_type=jnp.float32)
    h2 = h2 + b_ref[...]
    o_ref[...] = h2.astype(jnp.bfloat16)
    s = jnp.sum(h2, axis=0, keepdims=True)
    sq = jnp.sum(h2 * h2, axis=0, keepdims=True)
    st_out_ref[...] = jnp.concatenate([s, sq], axis=0)[None]


def _bn_mm_body(n_rows, h_ref, st_in_ref, g_ref, be_ref, w_ref, b_ref, o_ref):
    scale, shift = _bn_from_partials(st_in_ref, g_ref, be_ref, n_rows)
    a = jnp.maximum(h_ref[...].astype(jnp.float32) * scale + shift, 0.0)
    ab = a.astype(jnp.bfloat16)
    wb = w_ref[...].astype(jnp.bfloat16)
    h2 = jax.lax.dot_general(ab, wb, _DN, preferred_element_type=jnp.float32)
    o_ref[...] = h2 + b_ref[...]


def _row_spec(blk, d):
    return pl.BlockSpec((blk, d), lambda i: (i, 0))


def _full_spec(shape):
    nd = len(shape)
    return pl.BlockSpec(shape, lambda i: (0,) * nd)


def _part_spec(f):
    return pl.BlockSpec((1, 2, f), lambda i: (i, 0, 0))


def kernel(input, W1, b1, g1, be1, W2, b2, g2, be2, W3, b3):
    n, d = input.shape
    f = W1.shape[0]
    blk = 5000
    nblk = n // blk
    grid = (nblk,)
    params = pltpu.CompilerParams(dimension_semantics=("arbitrary",))

    b1r = b1.reshape(1, f)
    b2r = b2.reshape(1, f)
    b3r = b3.reshape(1, f)
    g1r = g1.reshape(1, f)
    g2r = g2.reshape(1, f)
    be1r = be1.reshape(1, f)
    be2r = be2.reshape(1, f)

    h1, st1 = pl.pallas_call(
        _mm_stats_body,
        grid=grid,
        in_specs=[_row_spec(blk, d), _full_spec((f, d)), _full_spec((1, f))],
        out_specs=[_row_spec(blk, f), _part_spec(f)],
        out_shape=[
            jax.ShapeDtypeStruct((n, f), jnp.bfloat16),
            jax.ShapeDtypeStruct((nblk, 2, f), jnp.float32),
        ],
        compiler_params=params,
    )(input, W1, b1r)

    h2, st2 = pl.pallas_call(
        functools.partial(_bn_mm_stats_body, float(n)),
        grid=grid,
        in_specs=[_row_spec(blk, f), _full_spec((nblk, 2, f)),
                  _full_spec((1, f)), _full_spec((1, f)),
                  _full_spec((f, f)), _full_spec((1, f))],
        out_specs=[_row_spec(blk, f), _part_spec(f)],
        out_shape=[
            jax.ShapeDtypeStruct((n, f), jnp.bfloat16),
            jax.ShapeDtypeStruct((nblk, 2, f), jnp.float32),
        ],
        compiler_params=params,
    )(h1, st1, g1r, be1r, W2, b2r)

    out = pl.pallas_call(
        functools.partial(_bn_mm_body, float(n)),
        grid=grid,
        in_specs=[_row_spec(blk, f), _full_spec((nblk, 2, f)),
                  _full_spec((1, f)), _full_spec((1, f)),
                  _full_spec((f, f)), _full_spec((1, f))],
        out_specs=_row_spec(blk, f),
        out_shape=jax.ShapeDtypeStruct((n, f), jnp.float32),
        compiler_params=params,
    )(h2, st2, g2r, be2r, W3, b3r)

    return out
